# baseline (device time: 8569 ns/iter reference)
import functools

import jax
import jax.numpy as jnp
from jax import lax
from jax.experimental import pallas as pl
from jax.experimental.pallas import tpu as pltpu

N_DEV = 16
KTAPS = 4
HALO = KTAPS - 1


def kernel(x, k):
    b, s, c = x.shape

    def body(x_ref, k_ref, out_ref, halo_ref, send_buf, send_sem, recv_sem):
        my_i = lax.axis_index("i")
        left = (my_i - 1) % N_DEV
        right = (my_i + 1) % N_DEV
        is_first = my_i == 0
        is_last = my_i == N_DEV - 1

        barrier_sem = pltpu.get_barrier_semaphore()

        @pl.when(jnp.logical_not(is_first))
        def _():
            pl.semaphore_signal(
                barrier_sem, inc=1,
                device_id=(left,), device_id_type=pl.DeviceIdType.MESH,
            )

        send_buf[...] = x_ref[:, pl.ds(s - HALO, HALO), :]

        rdma = pltpu.make_async_remote_copy(
            src_ref=send_buf,
            dst_ref=halo_ref,
            send_sem=send_sem,
            recv_sem=recv_sem,
            device_id=(right,),
            device_id_type=pl.DeviceIdType.MESH,
        )

        @pl.when(jnp.logical_not(is_last))
        def _():
            pl.semaphore_wait(barrier_sem, 1)
            rdma.start()

        x_val = x_ref[...].astype(jnp.bfloat16)
        kv = k_ref[...].astype(jnp.bfloat16)
        acc = x_val[:, HALO:, :] * kv[KTAPS - 1, :]
        for t in range(KTAPS - 1):
            acc = acc + x_val[:, t:t + s - HALO, :] * kv[t, :]
        out_ref[:, HALO:, :] = (
            acc / (1.0 + jnp.exp(-acc)).astype(jnp.bfloat16)
        ).astype(out_ref.dtype)

        @pl.when(jnp.logical_not(is_last))
        def _():
            rdma.wait_send()

        @pl.when(jnp.logical_not(is_first))
        def _():
            rdma.wait_recv()

        halo = jnp.where(
            is_first, jnp.zeros_like(halo_ref[...]), halo_ref[...]
        ).astype(jnp.bfloat16)
        pad = jnp.concatenate([halo, x_val[:, :HALO, :]], axis=1)
        acc0 = pad[:, HALO:, :] * kv[KTAPS - 1, :]
        for t in range(KTAPS - 1):
            acc0 = acc0 + pad[:, t:t + HALO, :] * kv[t, :]
        out_ref[:, :HALO, :] = (
            acc0 / (1.0 + jnp.exp(-acc0)).astype(jnp.bfloat16)
        ).astype(out_ref.dtype)

        @functools.partial(
            pl.run_scoped, credit_sem=pltpu.SemaphoreType.REGULAR
        )
        def _(credit_sem):
            @pl.when(jnp.logical_not(is_first))
            def _():
                pl.semaphore_signal(
                    credit_sem, inc=1,
                    device_id=(left,), device_id_type=pl.DeviceIdType.MESH,
                )

            @pl.when(jnp.logical_not(is_last))
            def _():
                pl.semaphore_wait(credit_sem, 1)

    return pl.pallas_call(
        body,
        out_shape=jax.ShapeDtypeStruct((b, s, c), x.dtype),
        in_specs=[
            pl.BlockSpec(memory_space=pltpu.VMEM),
            pl.BlockSpec(memory_space=pltpu.VMEM),
        ],
        out_specs=pl.BlockSpec(memory_space=pltpu.VMEM),
        scratch_shapes=[
            pltpu.VMEM((b, HALO, c), x.dtype),
            pltpu.VMEM((b, HALO, c), x.dtype),
            pltpu.SemaphoreType.DMA,
            pltpu.SemaphoreType.DMA,
        ],
        compiler_params=pltpu.CompilerParams(collective_id=0),
    )(x, k)


# device time: 8513 ns/iter; 1.0066x vs baseline; 1.0066x over previous
import functools

import jax
import jax.numpy as jnp
from jax import lax
from jax.experimental import pallas as pl
from jax.experimental.pallas import tpu as pltpu

N_DEV = 16
KTAPS = 4
HALO = KTAPS - 1


def kernel(x, k):
    b, s, c = x.shape

    def body(x_ref, k_ref, out_ref, halo_ref, send_buf, send_sem, recv_sem):
        my_i = lax.axis_index("i")
        left = (my_i - 1) % N_DEV
        right = (my_i + 1) % N_DEV
        is_first = my_i == 0
        is_last = my_i == N_DEV - 1

        barrier_sem = pltpu.get_barrier_semaphore()

        @pl.when(jnp.logical_not(is_first))
        def _():
            pl.semaphore_signal(
                barrier_sem, inc=1,
                device_id=(left,), device_id_type=pl.DeviceIdType.MESH,
            )

        send_buf[...] = x_ref[:, pl.ds(s - HALO, HALO), :]

        rdma = pltpu.make_async_remote_copy(
            src_ref=send_buf,
            dst_ref=halo_ref,
            send_sem=send_sem,
            recv_sem=recv_sem,
            device_id=(right,),
            device_id_type=pl.DeviceIdType.MESH,
        )

        @pl.when(jnp.logical_not(is_last))
        def _():
            pl.semaphore_wait(barrier_sem, 1)
            rdma.start()

        @pl.when(is_first)
        def _():
            halo_ref[...] = jnp.zeros((b, HALO, c), halo_ref.dtype)

        x_val = x_ref[...].astype(jnp.bfloat16)
        kv = k_ref[...].astype(jnp.bfloat16)
        acc = x_val[:, HALO:, :] * kv[KTAPS - 1, :]
        for t in range(KTAPS - 1):
            acc = acc + x_val[:, t:t + s - HALO, :] * kv[t, :]
        out_ref[:, HALO:, :] = (
            acc / (1.0 + jnp.exp(-acc)).astype(jnp.bfloat16)
        ).astype(out_ref.dtype)

        @functools.partial(
            pl.run_scoped, credit_sem=pltpu.SemaphoreType.REGULAR
        )
        def _(credit_sem):
            @pl.when(jnp.logical_not(is_first))
            def _():
                rdma.wait_recv()
                pl.semaphore_signal(
                    credit_sem, inc=1,
                    device_id=(left,), device_id_type=pl.DeviceIdType.MESH,
                )

            halo = halo_ref[...].astype(jnp.bfloat16)
            pad = jnp.concatenate([halo, x_val[:, :HALO, :]], axis=1)
            acc0 = pad[:, HALO:, :] * kv[KTAPS - 1, :]
            for t in range(KTAPS - 1):
                acc0 = acc0 + pad[:, t:t + HALO, :] * kv[t, :]
            out_ref[:, :HALO, :] = (
                acc0 / (1.0 + jnp.exp(-acc0)).astype(jnp.bfloat16)
            ).astype(out_ref.dtype)

            @pl.when(jnp.logical_not(is_last))
            def _():
                rdma.wait_send()
                pl.semaphore_wait(credit_sem, 1)

    return pl.pallas_call(
        body,
        out_shape=jax.ShapeDtypeStruct((b, s, c), x.dtype),
        in_specs=[
            pl.BlockSpec(memory_space=pltpu.VMEM),
            pl.BlockSpec(memory_space=pltpu.VMEM),
        ],
        out_specs=pl.BlockSpec(memory_space=pltpu.VMEM),
        scratch_shapes=[
            pltpu.VMEM((b, HALO, c), x.dtype),
            pltpu.VMEM((b, HALO, c), x.dtype),
            pltpu.SemaphoreType.DMA,
            pltpu.SemaphoreType.DMA,
        ],
        compiler_params=pltpu.CompilerParams(collective_id=0),
    )(x, k)


# device time: 8289 ns/iter; 1.0338x vs baseline; 1.0270x over previous
import functools

import jax
import jax.numpy as jnp
from jax import lax
from jax.experimental import pallas as pl
from jax.experimental.pallas import tpu as pltpu

N_DEV = 16
KTAPS = 4
HALO = KTAPS - 1
HEAD = 16


def kernel(x, k):
    b, s, c = x.shape

    def body(x_hbm, k_hbm, out_hbm, x_vmem, k_vmem, out_vmem, halo_ref,
             send_buf, local_sems, send_sem, recv_sem):
        my_i = lax.axis_index("i")
        left = (my_i - 1) % N_DEV
        right = (my_i + 1) % N_DEV
        is_first = my_i == 0
        is_last = my_i == N_DEV - 1

        barrier_sem = pltpu.get_barrier_semaphore()

        @pl.when(jnp.logical_not(is_first))
        def _():
            pl.semaphore_signal(
                barrier_sem, inc=1,
                device_id=(left,), device_id_type=pl.DeviceIdType.MESH,
            )

        tail_cp = pltpu.make_async_copy(
            x_hbm.at[:, pl.ds(s - HALO, HALO), :], send_buf, local_sems.at[0]
        )
        tail_cp.start()
        x_cp = pltpu.make_async_copy(x_hbm, x_vmem, local_sems.at[1])
        x_cp.start()
        k_cp = pltpu.make_async_copy(k_hbm, k_vmem, local_sems.at[2])
        k_cp.start()

        rdma = pltpu.make_async_remote_copy(
            src_ref=send_buf,
            dst_ref=halo_ref,
            send_sem=send_sem,
            recv_sem=recv_sem,
            device_id=(right,),
            device_id_type=pl.DeviceIdType.MESH,
        )

        tail_cp.wait()

        @pl.when(jnp.logical_not(is_last))
        def _():
            pl.semaphore_wait(barrier_sem, 1)
            rdma.start()

        @pl.when(is_first)
        def _():
            halo_ref[...] = jnp.zeros((b, HALO, c), halo_ref.dtype)

        x_cp.wait()
        k_cp.wait()

        x_val = x_vmem[...].astype(jnp.bfloat16)
        kv = k_vmem[...].astype(jnp.bfloat16)
        acc = x_val[:, HALO:, :] * kv[KTAPS - 1, :]
        for t in range(KTAPS - 1):
            acc = acc + x_val[:, t:t + s - HALO, :] * kv[t, :]
        out_vmem[:, HALO:, :] = acc / (1.0 + jnp.exp(-acc)).astype(jnp.bfloat16)

        out_cp_main = pltpu.make_async_copy(
            out_vmem.at[:, pl.ds(HEAD, s - HEAD), :],
            out_hbm.at[:, pl.ds(HEAD, s - HEAD), :],
            local_sems.at[3],
        )
        out_cp_main.start()

        @functools.partial(
            pl.run_scoped, credit_sem=pltpu.SemaphoreType.REGULAR
        )
        def _(credit_sem):
            @pl.when(jnp.logical_not(is_first))
            def _():
                rdma.wait_recv()
                pl.semaphore_signal(
                    credit_sem, inc=1,
                    device_id=(left,), device_id_type=pl.DeviceIdType.MESH,
                )

            halo = halo_ref[...].astype(jnp.bfloat16)
            pad = jnp.concatenate([halo, x_val[:, :HALO, :]], axis=1)
            acc0 = pad[:, HALO:, :] * kv[KTAPS - 1, :]
            for t in range(KTAPS - 1):
                acc0 = acc0 + pad[:, t:t + HALO, :] * kv[t, :]
            out_vmem[:, :HALO, :] = (
                acc0 / (1.0 + jnp.exp(-acc0)).astype(jnp.bfloat16)
            )

            out_cp_head = pltpu.make_async_copy(
                out_vmem.at[:, pl.ds(0, HEAD), :],
                out_hbm.at[:, pl.ds(0, HEAD), :],
                local_sems.at[4],
            )
            out_cp_head.start()
            out_cp_head.wait()
            out_cp_main.wait()

            @pl.when(jnp.logical_not(is_last))
            def _():
                rdma.wait_send()
                pl.semaphore_wait(credit_sem, 1)

    return pl.pallas_call(
        body,
        out_shape=jax.ShapeDtypeStruct((b, s, c), jnp.bfloat16),
        in_specs=[
            pl.BlockSpec(memory_space=pl.ANY),
            pl.BlockSpec(memory_space=pl.ANY),
        ],
        out_specs=pl.BlockSpec(memory_space=pl.ANY),
        scratch_shapes=[
            pltpu.VMEM((b, s, c), x.dtype),
            pltpu.VMEM(k.shape, k.dtype),
            pltpu.VMEM((b, s, c), jnp.bfloat16),
            pltpu.VMEM((b, HALO, c), x.dtype),
            pltpu.VMEM((b, HALO, c), x.dtype),
            pltpu.SemaphoreType.DMA((5,)),
            pltpu.SemaphoreType.DMA,
            pltpu.SemaphoreType.DMA,
        ],
        compiler_params=pltpu.CompilerParams(collective_id=0),
    )(x, k)
